# baseline (speedup 1.0000x reference)
import jax
import jax.numpy as jnp
from jax.experimental import pallas as pl
from jax.experimental.pallas import tpu as pltpu

_N = 1048576
_K = max(int(_N * 0.05), 1)
_ROWS = 8192
_COLS = 128
_NBLK = 16
_BLK = _ROWS // _NBLK
_LOC = 3.0
_ONE_BITS = 0x3F800000


def _body(x_ref, mask_ref, s_ref, sum_ref, bits_scr, lohi, acc):
    p = pl.program_id(0)

    @pl.when(p < _NBLK)
    def _sigmoid_phase():
        s = jax.nn.sigmoid(x_ref[...] + _LOC)
        s_ref[...] = s
        bits_scr[pl.ds(jnp.minimum(p, _NBLK - 1) * _BLK, _BLK), :] = (
            jax.lax.bitcast_convert_type(s, jnp.int32))
        blk_sum = jnp.sum(jnp.sum(s.reshape(8, _BLK // 8, _COLS), axis=1))
        prev = jnp.where(p == 0, 0.0, acc[0])
        acc[0] = prev + blk_sum

    @pl.when(p == _NBLK)
    def _bisect_phase():
        sbits = bits_scr[...].reshape(64, _ROWS // 64, _COLS)

        def count_above(t):
            part = jax.lax.shift_right_logical(t - sbits, 31)
            return jnp.sum(jnp.sum(part, axis=1))

        def step(_, carry):
            lo, hi = carry
            mid = (lo + hi) >> 1
            big = count_above(mid) >= _K
            return jnp.where(big, mid, lo), jnp.where(big, hi, mid)

        _, hi = jax.lax.fori_loop(
            0, 30, step, (jnp.int32(-1), jnp.int32(_ONE_BITS)))
        lohi[0] = hi
        sum_ref[...] = acc[0][None, None]

    @pl.when(p > _NBLK)
    def _mask_phase():
        b = jnp.minimum(p - _NBLK - 1, _NBLK - 1)
        blk = bits_scr[pl.ds(b * _BLK, _BLK), :]
        mask_ref[...] = (blk > lohi[0]).astype(jnp.float32)


def kernel(input_element):
    x2 = input_element.reshape(_ROWS, _COLS)
    nsteps = 2 * _NBLK + 1
    mask, s, ssum = pl.pallas_call(
        _body,
        grid=(nsteps,),
        in_specs=[
            pl.BlockSpec((_BLK, _COLS), lambda p: (jnp.minimum(p, _NBLK - 1), 0)),
        ],
        out_specs=(
            pl.BlockSpec(
                (_BLK, _COLS),
                lambda p: (jnp.where(p > _NBLK,
                                     jnp.minimum(p - _NBLK - 1, _NBLK - 1),
                                     0), 0)),
            pl.BlockSpec((_BLK, _COLS), lambda p: (jnp.minimum(p, _NBLK - 1), 0)),
            pl.BlockSpec((1, 1), lambda p: (0, 0)),
        ),
        out_shape=(
            jax.ShapeDtypeStruct((_ROWS, _COLS), jnp.float32),
            jax.ShapeDtypeStruct((_ROWS, _COLS), jnp.float32),
            jax.ShapeDtypeStruct((1, 1), jnp.float32),
        ),
        scratch_shapes=[
            pltpu.VMEM((_ROWS, _COLS), jnp.int32),
            pltpu.SMEM((1,), jnp.int32),
            pltpu.SMEM((1,), jnp.float32),
        ],
    )(x2)
    s_flat = s.reshape(_N)
    mean = (ssum[0, 0] / _N).astype(jnp.float32)
    return (mask.reshape(_N), mean, s_flat, s_flat)


# speedup vs baseline: 1.2855x; 1.2855x over previous
import jax
import jax.numpy as jnp
from jax.experimental import pallas as pl
from jax.experimental.pallas import tpu as pltpu

_N = 1048576
_K = max(int(_N * 0.05), 1)
_ROWS = 8192
_COLS = 128
_NBLK = 4
_BLK = _ROWS // _NBLK
_LOC = 3.0
_ONE_BITS = 0x3F800000


def _body(x_ref, mask_ref, s_ref, sum_ref, bits_scr, lohi, acc):
    p = pl.program_id(0)

    @pl.when(p < _NBLK)
    def _sigmoid_phase():
        s = jax.nn.sigmoid(x_ref[...] + _LOC)
        s_ref[...] = s
        bits_scr[pl.ds(jnp.minimum(p, _NBLK - 1) * _BLK, _BLK), :] = (
            jax.lax.bitcast_convert_type(s, jnp.int32))
        blk_sum = jnp.sum(jnp.sum(s.reshape(8, _BLK // 8, _COLS), axis=1))
        prev = jnp.where(p == 0, 0.0, acc[0])
        acc[0] = prev + blk_sum

    @pl.when(p == _NBLK)
    def _bisect_phase():
        sbits = bits_scr[...].reshape(64, _ROWS // 64, _COLS)

        def count_above(t):
            part = jax.lax.shift_right_logical(t - sbits, 31)
            return jnp.sum(jnp.sum(part, axis=1))

        def step(_, carry):
            lo, hi = carry
            mid = (lo + hi) >> 1
            big = count_above(mid) >= _K
            return jnp.where(big, mid, lo), jnp.where(big, hi, mid)

        _, hi = jax.lax.fori_loop(
            0, 30, step, (jnp.int32(-1), jnp.int32(_ONE_BITS)))
        lohi[0] = hi
        sum_ref[...] = acc[0][None, None]

    @pl.when(p > _NBLK)
    def _mask_phase():
        b = jnp.minimum(p - _NBLK - 1, _NBLK - 1)
        blk = bits_scr[pl.ds(b * _BLK, _BLK), :]
        mask_ref[...] = (blk > lohi[0]).astype(jnp.float32)


def kernel(input_element):
    x2 = input_element.reshape(_ROWS, _COLS)
    nsteps = 2 * _NBLK + 1
    mask, s, ssum = pl.pallas_call(
        _body,
        grid=(nsteps,),
        in_specs=[
            pl.BlockSpec((_BLK, _COLS), lambda p: (jnp.minimum(p, _NBLK - 1), 0)),
        ],
        out_specs=(
            pl.BlockSpec(
                (_BLK, _COLS),
                lambda p: (jnp.where(p > _NBLK,
                                     jnp.minimum(p - _NBLK - 1, _NBLK - 1),
                                     0), 0)),
            pl.BlockSpec((_BLK, _COLS), lambda p: (jnp.minimum(p, _NBLK - 1), 0)),
            pl.BlockSpec((1, 1), lambda p: (0, 0)),
        ),
        out_shape=(
            jax.ShapeDtypeStruct((_ROWS, _COLS), jnp.float32),
            jax.ShapeDtypeStruct((_ROWS, _COLS), jnp.float32),
            jax.ShapeDtypeStruct((1, 1), jnp.float32),
        ),
        scratch_shapes=[
            pltpu.VMEM((_ROWS, _COLS), jnp.int32),
            pltpu.SMEM((1,), jnp.int32),
            pltpu.SMEM((1,), jnp.float32),
        ],
    )(x2)
    s_flat = s.reshape(_N)
    mean = (ssum[0, 0] / _N).astype(jnp.float32)
    return (mask.reshape(_N), mean, s_flat, s_flat)


# speedup vs baseline: 1.3520x; 1.0517x over previous
import jax
import jax.numpy as jnp
from jax.experimental import pallas as pl
from jax.experimental.pallas import tpu as pltpu

_N = 1048576
_K = max(int(_N * 0.05), 1)
_ROWS = 8192
_COLS = 128
_NBLK = 2
_BLK = _ROWS // _NBLK
_LOC = 3.0
_ONE_BITS = 0x3F800000


def _body(x_ref, mask_ref, s_ref, sum_ref, bits_scr, lohi, acc):
    p = pl.program_id(0)

    @pl.when(p < _NBLK)
    def _sigmoid_phase():
        s = jax.nn.sigmoid(x_ref[...] + _LOC)
        s_ref[...] = s
        bits_scr[pl.ds(jnp.minimum(p, _NBLK - 1) * _BLK, _BLK), :] = (
            jax.lax.bitcast_convert_type(s, jnp.int32))
        blk_sum = jnp.sum(jnp.sum(s.reshape(8, _BLK // 8, _COLS), axis=1))
        prev = jnp.where(p == 0, 0.0, acc[0])
        acc[0] = prev + blk_sum

    @pl.when(p == _NBLK)
    def _bisect_phase():
        sbits = bits_scr[...].reshape(64, _ROWS // 64, _COLS)

        def count_above(t):
            part = jax.lax.shift_right_logical(t - sbits, 31)
            return jnp.sum(jnp.sum(part, axis=1))

        def step(_, carry):
            lo, hi = carry
            mid = (lo + hi) >> 1
            big = count_above(mid) >= _K
            return jnp.where(big, mid, lo), jnp.where(big, hi, mid)

        _, hi = jax.lax.fori_loop(
            0, 30, step, (jnp.int32(-1), jnp.int32(_ONE_BITS)))
        lohi[0] = hi
        sum_ref[...] = acc[0][None, None]

    @pl.when(p > _NBLK)
    def _mask_phase():
        b = jnp.minimum(p - _NBLK - 1, _NBLK - 1)
        blk = bits_scr[pl.ds(b * _BLK, _BLK), :]
        mask_ref[...] = (blk > lohi[0]).astype(jnp.float32)


def kernel(input_element):
    x2 = input_element.reshape(_ROWS, _COLS)
    nsteps = 2 * _NBLK + 1
    mask, s, ssum = pl.pallas_call(
        _body,
        grid=(nsteps,),
        in_specs=[
            pl.BlockSpec((_BLK, _COLS), lambda p: (jnp.minimum(p, _NBLK - 1), 0)),
        ],
        out_specs=(
            pl.BlockSpec(
                (_BLK, _COLS),
                lambda p: (jnp.where(p > _NBLK,
                                     jnp.minimum(p - _NBLK - 1, _NBLK - 1),
                                     0), 0)),
            pl.BlockSpec((_BLK, _COLS), lambda p: (jnp.minimum(p, _NBLK - 1), 0)),
            pl.BlockSpec((1, 1), lambda p: (0, 0)),
        ),
        out_shape=(
            jax.ShapeDtypeStruct((_ROWS, _COLS), jnp.float32),
            jax.ShapeDtypeStruct((_ROWS, _COLS), jnp.float32),
            jax.ShapeDtypeStruct((1, 1), jnp.float32),
        ),
        scratch_shapes=[
            pltpu.VMEM((_ROWS, _COLS), jnp.int32),
            pltpu.SMEM((1,), jnp.int32),
            pltpu.SMEM((1,), jnp.float32),
        ],
    )(x2)
    s_flat = s.reshape(_N)
    mean = (ssum[0, 0] / _N).astype(jnp.float32)
    return (mask.reshape(_N), mean, s_flat, s_flat)
